# baseline (speedup 1.0000x reference)
import functools

import jax
import jax.numpy as jnp
import numpy as np
from jax import lax
from jax.experimental import pallas as pl
from jax.experimental.pallas import tpu as pltpu
from jax.experimental.pallas import tpu_sc as plsc

VOCAB = 100000
D = 64
B = 16384

NC = 2
NS = 16
NW = NC * NS
BPW = B // NW

_EPS = 1e-8


def _sc_gather_pair():
    mesh = plsc.VectorSubcoreMesh(core_axis_name="c", subcore_axis_name="s")

    @functools.partial(
        pl.kernel,
        out_type=(
            jax.ShapeDtypeStruct((B, D), jnp.float32),
            jax.ShapeDtypeStruct((B, D), jnp.float32),
        ),
        mesh=mesh,
        scratch_types=[
            pltpu.VMEM((BPW,), jnp.int32),
            pltpu.VMEM((BPW, D), jnp.float32),
            pltpu.VMEM((BPW,), jnp.int32),
            pltpu.VMEM((BPW, D), jnp.float32),
            pltpu.SemaphoreType.DMA,
            pltpu.SemaphoreType.DMA,
        ],
        compiler_params=pltpu.CompilerParams(use_tc_tiling_on_sc=False),
    )
    def gather2(t1, idx1_hbm, t2, idx2_hbm, out1, out2,
                idx1_v, rows1_v, idx2_v, rows2_v, sem1, sem2):
        wid = lax.axis_index("s") * NC + lax.axis_index("c")
        base = wid * BPW
        pltpu.sync_copy(idx1_hbm.at[pl.ds(base, BPW)], idx1_v)
        pltpu.sync_copy(idx2_hbm.at[pl.ds(base, BPW)], idx2_v)
        c1 = pltpu.async_copy(t1.at[idx1_v], rows1_v, sem1)
        c2 = pltpu.async_copy(t2.at[idx2_v], rows2_v, sem2)
        c1.wait()
        c2.wait()
        pltpu.sync_copy(rows1_v, out1.at[pl.ds(base, BPW)])
        pltpu.sync_copy(rows2_v, out2.at[pl.ds(base, BPW)])

    return gather2


def _tc_body(e1_ref, e2_ref, m_ref, o_ref):
    m = m_ref[...]
    w1 = jnp.dot(e1_ref[...], m, preferred_element_type=jnp.float32)
    w2 = jnp.dot(e2_ref[...], m, preferred_element_type=jnp.float32)
    num = jnp.sum(w1 * w2, axis=1)
    n1 = jnp.sum(w1 * w1, axis=1)
    n2 = jnp.sum(w2 * w2, axis=1)
    o_ref[...] = num / (jnp.sqrt(n1) * jnp.sqrt(n2) + _EPS)


_BB = 2048


def _tc_bind_cosine(e1, e2, m):
    grid = B // _BB
    return pl.pallas_call(
        _tc_body,
        grid=(grid,),
        in_specs=[
            pl.BlockSpec((_BB, D), lambda i: (i, 0)),
            pl.BlockSpec((_BB, D), lambda i: (i, 0)),
            pl.BlockSpec((D, D), lambda i: (0, 0)),
        ],
        out_specs=pl.BlockSpec((_BB,), lambda i: (i,)),
        out_shape=jax.ShapeDtypeStruct((B,), jnp.float32),
    )(e1, e2, m)


_CIRC_P = np.zeros((D * D, D), np.float32)
for _j in range(D):
    for _k in range(D):
        _CIRC_P[_j * D + _k, (_k - _j) % D] = 1.0


def _circulant(d):
    p = jnp.asarray(_CIRC_P)
    return lax.dot_general(p, d, (((1,), (0,)), ((), ())),
                           precision=lax.Precision.HIGHEST).reshape(D, D)


def kernel(source1, source2, dummy_vector, word1, word2):
    idx1 = word1.astype(jnp.int32)
    idx2 = word2.astype(jnp.int32)
    e1, e2 = _sc_gather_pair()(source1, idx1, source2, idx2)
    m = _circulant(dummy_vector)
    return _tc_bind_cosine(e1, e2, m)


# speedup vs baseline: 1.2914x; 1.2914x over previous
import functools

import jax
import jax.numpy as jnp
import numpy as np
from jax import lax
from jax.experimental import pallas as pl
from jax.experimental.pallas import tpu as pltpu
from jax.experimental.pallas import tpu_sc as plsc

VOCAB = 100000
D = 64
B = 16384

NC = 2
NS = 16
NW = NC * NS
BPW = B // NW

_EPS = 1e-8

_SLAB = 4096
_NSLAB = (VOCAB + _SLAB - 1) // _SLAB
_H = _SLAB // 2
VP = _NSLAB * _H
_LOG_SLAB = _SLAB.bit_length() - 1
_LOG_H = _H.bit_length() - 1


def _pack_body(a_ref, o_ref):
    a = a_ref[...]
    o_ref[...] = jnp.concatenate([a[:, :_H].T, a[:, _H:].T], axis=1)


def _relayout_pack(tt):
    return pl.pallas_call(
        _pack_body,
        grid=(_NSLAB,),
        in_specs=[pl.BlockSpec((D, _SLAB), lambda i: (0, i))],
        out_specs=pl.BlockSpec((_H, 128), lambda i: (i, 0)),
        out_shape=jax.ShapeDtypeStruct((VP, 128), jnp.float32),
    )(tt)


def _sc_gather_one():
    mesh = plsc.VectorSubcoreMesh(core_axis_name="c", subcore_axis_name="s")

    @functools.partial(
        pl.kernel,
        out_type=jax.ShapeDtypeStruct((B, 128), jnp.float32),
        mesh=mesh,
        scratch_types=[
            pltpu.VMEM((BPW,), jnp.int32),
            pltpu.VMEM((BPW,), jnp.int32),
            pltpu.VMEM((BPW, 128), jnp.float32),
            pltpu.SemaphoreType.DMA,
        ],
        compiler_params=pltpu.CompilerParams(use_tc_tiling_on_sc=True),
    )
    def gather1(tp, idx_hbm, out, idx_v, sidx_v, rows_v, sem):
        wid = lax.axis_index("s") * NC + lax.axis_index("c")
        base = wid * BPW
        pltpu.sync_copy(idx_hbm.at[pl.ds(base, BPW)], idx_v)
        for j in range(BPW // 16):
            s = pl.ds(16 * j, 16)
            r = idx_v[s]
            sidx_v[s] = (lax.shift_left(lax.shift_right_logical(r, _LOG_SLAB),
                                        _LOG_H)
                         | (r & (_H - 1)))
        pltpu.async_copy(tp.at[sidx_v], rows_v, sem).wait()
        pltpu.sync_copy(rows_v, out.at[pl.ds(base, BPW)])

    return gather1


def _tc_body(e1p_ref, e2p_ref, p1_ref, p2_ref, m_ref, o_ref):
    m = m_ref[...]
    odd1 = p1_ref[...].T > 0.5
    odd2 = p2_ref[...].T > 0.5
    e1 = jnp.where(odd1, e1p_ref[:, 64:], e1p_ref[:, :64])
    e2 = jnp.where(odd2, e2p_ref[:, 64:], e2p_ref[:, :64])
    w1 = jnp.dot(e1, m, preferred_element_type=jnp.float32)
    w2 = jnp.dot(e2, m, preferred_element_type=jnp.float32)
    num = jnp.sum(w1 * w2, axis=1)
    n1 = jnp.sum(w1 * w1, axis=1)
    n2 = jnp.sum(w2 * w2, axis=1)
    o_ref[...] = num / (jnp.sqrt(n1) * jnp.sqrt(n2) + _EPS)


_BB = 2048


def _tc_bind_cosine(e1p, e2p, idx1, idx2, m):
    grid = B // _BB
    par1 = (lax.shift_right_logical(idx1, _LOG_H) & 1).astype(
        jnp.float32)[None, :]
    par2 = (lax.shift_right_logical(idx2, _LOG_H) & 1).astype(
        jnp.float32)[None, :]
    return pl.pallas_call(
        _tc_body,
        grid=(grid,),
        in_specs=[
            pl.BlockSpec((_BB, 128), lambda i: (i, 0)),
            pl.BlockSpec((_BB, 128), lambda i: (i, 0)),
            pl.BlockSpec((1, _BB), lambda i: (0, i)),
            pl.BlockSpec((1, _BB), lambda i: (0, i)),
            pl.BlockSpec((D, D), lambda i: (0, 0)),
        ],
        out_specs=pl.BlockSpec((_BB,), lambda i: (i,)),
        out_shape=jax.ShapeDtypeStruct((B,), jnp.float32),
    )(e1p, e2p, par1, par2, m)


_CIRC_P = np.zeros((D * D, D), np.float32)
for _j in range(D):
    for _k in range(D):
        _CIRC_P[_j * D + _k, (_k - _j) % D] = 1.0


def _circulant(d):
    p = jnp.asarray(_CIRC_P)
    return lax.dot_general(p, d, (((1,), (0,)), ((), ())),
                           precision=lax.Precision.HIGHEST).reshape(D, D)


def kernel(source1, source2, dummy_vector, word1, word2):
    idx1 = word1.astype(jnp.int32)
    idx2 = word2.astype(jnp.int32)
    g = _sc_gather_one()
    t1p = _relayout_pack(source1.T)
    e1p = g(t1p, idx1)
    t2p = _relayout_pack(source2.T)
    e2p = g(t2p, idx2)
    m = _circulant(dummy_vector)
    return _tc_bind_cosine(e1p, e2p, idx1, idx2, m)


# speedup vs baseline: 1.3322x; 1.0316x over previous
import functools

import jax
import jax.numpy as jnp
import numpy as np
from jax import lax
from jax.experimental import pallas as pl
from jax.experimental.pallas import tpu as pltpu
from jax.experimental.pallas import tpu_sc as plsc

VOCAB = 100000
D = 64
B = 16384

NC = 2
NS = 16
NW = NC * NS
BPW = B // NW

_EPS = 1e-8

_SLAB = 4096
_NSLAB = (VOCAB + _SLAB - 1) // _SLAB
_H = _SLAB // 2
VP = _NSLAB * _H
_LOG_SLAB = _SLAB.bit_length() - 1
_LOG_H = _H.bit_length() - 1


def _pack_body(a_ref, o_ref):
    a = a_ref[...]
    o_ref[...] = jnp.concatenate([a[:, :_H].T, a[:, _H:].T], axis=1)


def _relayout_pack(tt):
    return pl.pallas_call(
        _pack_body,
        grid=(_NSLAB,),
        in_specs=[pl.BlockSpec((D, _SLAB), lambda i: (0, i))],
        out_specs=pl.BlockSpec((_H, 128), lambda i: (i, 0)),
        out_shape=jax.ShapeDtypeStruct((VP, 128), jnp.float32),
    )(tt)


def _sc_gather_one():
    mesh = plsc.VectorSubcoreMesh(core_axis_name="c", subcore_axis_name="s")

    @functools.partial(
        pl.kernel,
        out_type=jax.ShapeDtypeStruct((B, 128), jnp.float32),
        mesh=mesh,
        scratch_types=[
            pltpu.VMEM((BPW,), jnp.int32),
            pltpu.VMEM((BPW,), jnp.int32),
            pltpu.VMEM((BPW, 128), jnp.float32),
            pltpu.SemaphoreType.DMA,
        ],
        compiler_params=pltpu.CompilerParams(use_tc_tiling_on_sc=True),
    )
    def gather1(tp, idx_hbm, out, idx_v, sidx_v, rows_v, sem):
        wid = lax.axis_index("s") * NC + lax.axis_index("c")
        base = wid * BPW
        pltpu.sync_copy(idx_hbm.at[pl.ds(base, BPW)], idx_v)
        for j in range(BPW // 16):
            s = pl.ds(16 * j, 16)
            r = idx_v[s]
            sidx_v[s] = (lax.shift_left(lax.shift_right_logical(r, _LOG_SLAB),
                                        _LOG_H)
                         | (r & (_H - 1)))
        pltpu.async_copy(tp.at[sidx_v], rows_v, sem).wait()
        pltpu.sync_copy(rows_v, out.at[pl.ds(base, BPW)])

    return gather1


def _tc_body(e1p_ref, e2p_ref, p1_ref, p2_ref, m_ref, o_ref):
    m = m_ref[...]
    i = pl.program_id(0)
    nb = B // _BB
    oh = (lax.broadcasted_iota(jnp.int32, (_BB, nb), 1) == i).astype(
        jnp.float32)
    odd1 = jnp.sum(p1_ref[...] * oh, axis=1, keepdims=True) > 0.5
    odd2 = jnp.sum(p2_ref[...] * oh, axis=1, keepdims=True) > 0.5
    e1 = jnp.where(odd1, e1p_ref[:, 64:], e1p_ref[:, :64])
    e2 = jnp.where(odd2, e2p_ref[:, 64:], e2p_ref[:, :64])
    w1 = jnp.dot(e1, m, preferred_element_type=jnp.float32)
    w2 = jnp.dot(e2, m, preferred_element_type=jnp.float32)
    num = jnp.sum(w1 * w2, axis=1)
    n1 = jnp.sum(w1 * w1, axis=1)
    n2 = jnp.sum(w2 * w2, axis=1)
    o_ref[...] = num / (jnp.sqrt(n1) * jnp.sqrt(n2) + _EPS)


_BB = 2048


def _tc_bind_cosine(e1p, e2p, idx1, idx2, m):
    grid = B // _BB
    par1 = (lax.shift_right_logical(idx1, _LOG_H) & 1).astype(
        jnp.float32).reshape(grid, _BB).T
    par2 = (lax.shift_right_logical(idx2, _LOG_H) & 1).astype(
        jnp.float32).reshape(grid, _BB).T
    return pl.pallas_call(
        _tc_body,
        grid=(grid,),
        in_specs=[
            pl.BlockSpec((_BB, 128), lambda i: (i, 0)),
            pl.BlockSpec((_BB, 128), lambda i: (i, 0)),
            pl.BlockSpec((_BB, grid), lambda i: (0, 0)),
            pl.BlockSpec((_BB, grid), lambda i: (0, 0)),
            pl.BlockSpec((D, D), lambda i: (0, 0)),
        ],
        out_specs=pl.BlockSpec((_BB,), lambda i: (i,)),
        out_shape=jax.ShapeDtypeStruct((B,), jnp.float32),
    )(e1p, e2p, par1, par2, m)


_CIRC_P = np.zeros((D * D, D), np.float32)
for _j in range(D):
    for _k in range(D):
        _CIRC_P[_j * D + _k, (_k - _j) % D] = 1.0


def _circulant(d):
    p = jnp.asarray(_CIRC_P)
    return lax.dot_general(p, d, (((1,), (0,)), ((), ())),
                           precision=lax.Precision.HIGHEST).reshape(D, D)


def kernel(source1, source2, dummy_vector, word1, word2):
    idx1 = word1.astype(jnp.int32)
    idx2 = word2.astype(jnp.int32)
    g = _sc_gather_one()
    t1p = _relayout_pack(source1.T)
    e1p = g(t1p, idx1)
    t2p = _relayout_pack(source2.T)
    e2p = g(t2p, idx2)
    m = _circulant(dummy_vector)
    return _tc_bind_cosine(e1p, e2p, idx1, idx2, m)
